# baseline (speedup 1.0000x reference)
import jax
import jax.numpy as jnp
from jax import lax
from jax.experimental import pallas as pl
from jax.experimental.pallas import tpu as pltpu
from jax.experimental.pallas import tpu_sc as plsc

_B, _T, _H, _W = 1024, 8, 16, 16
_J = _T * _H * _W
_NE = 64
_VOCAB = 10
_NC, _NS, _L = 2, 16, 16
_NW = _NC * _NS
_SPW = _B // _NW
_BW = 768
_VPS = _J // _L


def _scatter_body(x_hbm, bins_hbm, xall, bins):
    cid = lax.axis_index("c")
    sid = lax.axis_index("s")
    wid = sid * _NC + cid
    base = wid * _SPW
    pltpu.sync_copy(x_hbm.at[pl.ds(base * _J, _SPW * _J)], xall)
    lane = lax.iota(jnp.int32, _L)
    lane16 = lane * _L
    ones = jnp.ones((_L,), jnp.float32)
    zeros16 = jnp.zeros((_L,), jnp.float32)

    @plsc.parallel_loop(0, _SPW * _BW // _L, step=16)
    def _(z):
        zbase = pl.multiple_of(z * _L, _L * 16)
        for k in range(16):
            bins[pl.ds(zbase + k * _L, _L)] = zeros16

    @plsc.parallel_loop(0, _SPW * _VPS, step=8)
    def _(i):
        s = i // _VPS
        im16 = i % _H
        vbase = lane16 + s * _BW
        rbase = vbase + (_L * _L) + im16
        cbase = lane + (s * _BW + 2 * _L * _L)
        off = pl.multiple_of(i * _L, _L * 8)
        for k in range(8):
            xv = xall[pl.ds(off + k * _L, _L)]
            maskf = jnp.minimum(xv, 1).astype(jnp.float32)
            plsc.addupdate_scatter(bins, [vbase + xv], ones)
            plsc.addupdate_scatter(bins, [rbase + k], maskf)
            plsc.addupdate_scatter(bins, [cbase], maskf)

    pltpu.sync_copy(bins, bins_hbm.at[pl.ds(base * _BW, _SPW * _BW)])


def _sc_scatter(x2):
    mesh = plsc.VectorSubcoreMesh(core_axis_name="c", subcore_axis_name="s",
                                  num_cores=_NC, num_subcores=_NS)
    fn = pl.kernel(
        _scatter_body,
        out_type=jax.ShapeDtypeStruct((_B * _BW,), jnp.float32),
        mesh=mesh,
        compiler_params=pltpu.CompilerParams(needs_layout_passes=False),
        scratch_types=[
            pltpu.VMEM((_SPW * _J,), jnp.int32),
            pltpu.VMEM((_SPW * _BW,), jnp.float32),
        ],
    )
    return fn(x2)


def _combine_body(bins_ref, hp_ref, row_ref, cole_ref, val_ref, w_ref,
                  b_ref, out_ref):
    bins = bins_ref[...]
    kk = lax.broadcasted_iota(jnp.int32, (_BW, 3 * _L), 0)
    vv = lax.broadcasted_iota(jnp.int32, (_BW, 3 * _L), 1)
    m16 = kk % _L
    val_s = ((m16 == vv).astype(jnp.float32)
             * (kk < _L * _L).astype(jnp.float32))
    row_s = ((m16 == (vv - _L)).astype(jnp.float32)
             * ((kk >= _L * _L) & (kk < 2 * _L * _L)).astype(jnp.float32))
    col_s = (((kk - 2 * _L * _L) == (vv - 2 * _L)).astype(jnp.float32)
             * (kk >= 2 * _L * _L).astype(jnp.float32))
    sel = val_s + row_s + col_s
    counts = jnp.dot(bins, sel, preferred_element_type=jnp.float32)
    valcnt = counts[:, :_L]
    rowcnt = counts[:, _L:2 * _L]
    colcnt = counts[:, 2 * _L:]

    vmask = (lax.broadcasted_iota(jnp.int32, (_VOCAB, 1), 0) != 0
             ).astype(jnp.float32)
    vtab = jnp.concatenate(
        [val_ref[...] * vmask, jnp.zeros((_L - _VOCAB, _NE), jnp.float32)],
        axis=0)
    num = (jnp.dot(valcnt, vtab, preferred_element_type=jnp.float32)
           + jnp.dot(rowcnt, row_ref[...], preferred_element_type=jnp.float32)
           + jnp.dot(colcnt, cole_ref[...],
                     preferred_element_type=jnp.float32))
    den = jnp.maximum(float(_J) - valcnt[:, 0:1], 1.0)
    h = num / den
    dn = (((1,), (1,)), ((), ()))
    out = lax.dot_general(h, w_ref[:, :_NE], dn,
                          preferred_element_type=jnp.float32)
    out = out + lax.dot_general(hp_ref[...], w_ref[:, _NE:], dn,
                                preferred_element_type=jnp.float32)
    out_ref[...] = out + b_ref[...]


@jax.jit
def kernel(x, h_parent, row_embed, col_embed, val_embed, head_w, head_b):
    x2 = x.reshape(_B * _J).astype(jnp.int32)
    bins = _sc_scatter(x2)
    nd = head_w.shape[0]
    out = pl.pallas_call(
        _combine_body,
        out_shape=jax.ShapeDtypeStruct((_B, nd), jnp.float32),
    )(bins.reshape(_B, _BW), h_parent, row_embed, col_embed, val_embed,
      head_w, head_b.reshape(1, -1))
    return out
